# speedup vs baseline: 2.7685x; 2.7685x over previous
import jax
import jax.numpy as jnp
from jax.experimental import pallas as pl


def _rows_kernel(table_ref, out_ref):
    out_ref[...] = table_ref[...]


def kernel(inputs, pos_table):
    seq_len = inputs.shape[-1]
    _, embed_dim = pos_table.shape
    block = 512
    grid = (seq_len // block,)
    return pl.pallas_call(
        _rows_kernel,
        grid=grid,
        in_specs=[pl.BlockSpec((block, embed_dim), lambda i: (i, 0))],
        out_specs=pl.BlockSpec((block, embed_dim), lambda i: (i, 0)),
        out_shape=jax.ShapeDtypeStruct((seq_len, embed_dim), pos_table.dtype),
    )(pos_table)
